# Initial kernel scaffold; baseline (speedup 1.0000x reference)
import jax
import jax.numpy as jnp
from jax.experimental import pallas as pl


def kernel(x, edge_index, batch, W1, b1, W2, b2, fc_W, fc_b):
    raise NotImplementedError("write your pallas kernel here")


# speedup vs baseline: 9.1612x; 9.1612x over previous
import functools

import jax
import jax.numpy as jnp
from jax import lax
from jax.experimental import pallas as pl
from jax.experimental.pallas import tpu as pltpu
from jax.experimental.pallas import tpu_sc as plsc

_NC, _NS = 2, 16
_N = 10000
_E = 160000
_NPAD = 10240
_RPS = _NPAD // _NS
_EPAD = 163840
_NSTR = 80
_DSTR = 40
_FC = 128
_BM = 1024
_G = 16
_HALF = _NPAD // 2
_HRPS = _HALF // _NS
_ACCR = _HALF + 16


def _sc_mesh():
    return plsc.VectorSubcoreMesh(
        core_axis_name="c", subcore_axis_name="s",
        num_cores=_NC, num_subcores=_NS)


@functools.partial(
    pl.kernel,
    out_type=(jax.ShapeDtypeStruct((_NPAD,), jnp.float32),
              jax.ShapeDtypeStruct((_NPAD,), jnp.float32)),
    mesh=_sc_mesh(),
    scratch_types=[
        pltpu.VMEM((_DSTR, 128), jnp.int32),
        pltpu.VMEM((128,), jnp.float32),
        pltpu.VMEM_SHARED((_NPAD,), jnp.float32),
        pltpu.SemaphoreType.DMA,
    ],
)
def _deg_kernel(dst_hbm, zeros_hbm, deg0_hbm, deg1_hbm, dst_v, ones_v, acc, sem):
    c = lax.axis_index("c")
    s = lax.axis_index("s")
    w = c * _NS + s
    pltpu.sync_copy(dst_hbm.at[w], dst_v)
    for i in range(8):
        ones_v[pl.ds(16 * i, 16)] = jnp.ones((16,), jnp.float32)
    pltpu.sync_copy(zeros_hbm, acc.at[pl.ds(s * _RPS, _RPS)])
    plsc.subcore_barrier()

    def blk(o, carry):
        def fire(i, carry2):
            pltpu.async_copy(ones_v, acc.at[dst_v.at[o * 8 + i]], sem, add=True)
            return carry2

        lax.fori_loop(0, 8, fire, 0)

        def drain(i, carry2):
            pltpu.make_async_copy(ones_v, acc.at[dst_v.at[0]], sem).wait()
            return carry2

        lax.fori_loop(0, 8, drain, 0)
        return carry

    lax.fori_loop(0, _DSTR // 8, blk, 0)
    plsc.subcore_barrier()

    @pl.when(c == 0)
    def _():
        pltpu.sync_copy(acc.at[pl.ds(s * _RPS, _RPS)],
                        deg0_hbm.at[pl.ds(s * _RPS, _RPS)])

    @pl.when(c == 1)
    def _():
        pltpu.sync_copy(acc.at[pl.ds(s * _RPS, _RPS)],
                        deg1_hbm.at[pl.ds(s * _RPS, _RPS)])


def _run_chunk(t, o, src_v, dst_v, bufs, acc, zeros_hbm, gsems, ssems, s, base):
    pltpu.sync_copy(zeros_hbm, acc.at[pl.ds(s * _HRPS, _HRPS)])
    plsc.subcore_barrier()
    for j in range(3):
        pltpu.async_copy(t.at[src_v.at[j]], bufs[j], gsems[j])

    def body(i, carry):
        for b in range(4):
            j = 4 * i + b
            pltpu.make_async_copy(t.at[src_v.at[0]], bufs[b], gsems[b]).wait()
            pltpu.async_copy(bufs[b], acc.at[dst_v.at[j]], ssems[b], add=True)
            pb = (b - 1) % 4
            if b == 0:
                @pl.when(i >= 1)
                def _():
                    pltpu.make_async_copy(
                        bufs[pb], acc.at[dst_v.at[0]], ssems[pb]).wait()
            else:
                pltpu.make_async_copy(
                    bufs[pb], acc.at[dst_v.at[0]], ssems[pb]).wait()
            gb = (b + 3) % 4
            if b == 0:
                pltpu.async_copy(t.at[src_v.at[j + 3]], bufs[gb], gsems[gb])
            else:
                @pl.when(i < _NSTR // 4 - 1)
                def _():
                    pltpu.async_copy(t.at[src_v.at[j + 3]], bufs[gb], gsems[gb])
        return carry

    lax.fori_loop(0, _NSTR // 4, body, 0)
    pltpu.make_async_copy(bufs[3], acc.at[dst_v.at[0]], ssems[3]).wait()
    plsc.subcore_barrier()
    pltpu.sync_copy(acc.at[pl.ds(s * _HRPS, _HRPS)],
                    o.at[pl.ds(base + s * _HRPS, _HRPS)])
    plsc.subcore_barrier()


_AGG_SCRATCH = ([pltpu.VMEM((_NSTR, 128), jnp.int32),
                 pltpu.VMEM((_NSTR, 128), jnp.int32)]
                + [pltpu.VMEM((128, _FC), jnp.float32) for _ in range(4)]
                + [pltpu.VMEM_SHARED((_ACCR, _FC), jnp.float32)]
                + [pltpu.SemaphoreType.DMA for _ in range(8)])


@functools.partial(
    pl.kernel,
    out_type=tuple(jax.ShapeDtypeStruct((_NPAD, _FC), jnp.float32)
                   for _ in range(4)),
    mesh=_sc_mesh(),
    scratch_types=_AGG_SCRATCH,
)
def _agg4(t0, t1, t2, t3, src_hbm, dst_hbm, zeros_hbm, o0, o1, o2, o3,
          src_v, dst_v, b0, b1, b2, b3, acc, g0, g1, g2, g3, s0, s1, s2, s3):
    tables = (t0, t1, t2, t3)
    out_refs = (o0, o1, o2, o3)
    bufs = (b0, b1, b2, b3)
    gsems = (g0, g1, g2, g3)
    ssems = (s0, s1, s2, s3)
    c = lax.axis_index("c")
    s = lax.axis_index("s")
    w = c * _NS + s
    base = c * _HALF
    pltpu.sync_copy(src_hbm.at[s], src_v)
    pltpu.sync_copy(dst_hbm.at[w], dst_v)
    for k in range(4):
        _run_chunk(tables[k], out_refs[k], src_v, dst_v, bufs,
                   acc, zeros_hbm, gsems, ssems, s, base)


def _tc_scale(x_pad, deg0, deg1):
    grid = _NPAD // _BM

    def body(x_ref, d0_ref, d1_ref, dinv_ref, t0_ref, t1_ref):
        deg = d0_ref[...] + d1_ref[...] + 1.0
        dinv = lax.rsqrt(deg)
        dinv_ref[...] = dinv
        t = x_ref[...] * dinv
        t0_ref[...] = t[:, :_FC]
        t1_ref[...] = t[:, _FC:]

    return pl.pallas_call(
        body,
        grid=(grid,),
        in_specs=[pl.BlockSpec((_BM, 256), lambda i: (i, 0)),
                  pl.BlockSpec((_BM, 1), lambda i: (i, 0)),
                  pl.BlockSpec((_BM, 1), lambda i: (i, 0))],
        out_specs=[pl.BlockSpec((_BM, 1), lambda i: (i, 0))]
        + [pl.BlockSpec((_BM, _FC), lambda i: (i, 0))] * 2,
        out_shape=[jax.ShapeDtypeStruct((_NPAD, 1), jnp.float32)]
        + [jax.ShapeDtypeStruct((_NPAD, _FC), jnp.float32)] * 2,
    )(x_pad, deg0, deg1)


def _tc_layer(aggs, ts, dinv, W, br):
    grid = _NPAD // _BM

    def body(*refs):
        ar = refs[0:4]
        tr = refs[4:8]
        dr, wr, brr = refs[8], refs[9], refs[10]
        houts = refs[11:15]
        touts = refs[15:19]
        dv = dr[...]
        z = jnp.concatenate([a[...] + t[...] for a, t in zip(ar, tr)],
                            axis=1) * dv
        h = lax.dot_general(z, wr[...], (((1,), (0,)), ((), ())),
                            preferred_element_type=jnp.float32)
        h = jnp.maximum(h + brr[...], 0.0)
        tn = h * dv
        for k in range(4):
            houts[k][...] = h[:, _FC * k:_FC * (k + 1)]
            touts[k][...] = tn[:, _FC * k:_FC * (k + 1)]

    cspec = pl.BlockSpec((_BM, _FC), lambda i: (i, 0))
    return pl.pallas_call(
        body,
        grid=(grid,),
        in_specs=[cspec] * 8 + [
            pl.BlockSpec((_BM, 1), lambda i: (i, 0)),
            pl.BlockSpec((512, 512), lambda i: (0, 0)),
            pl.BlockSpec((1, 512), lambda i: (0, 0))],
        out_specs=[cspec] * 8,
        out_shape=[jax.ShapeDtypeStruct((_NPAD, _FC), jnp.float32)
                   for _ in range(8)],
    )(*aggs, *ts, dinv, W, br)


def _tc_head(h2s, batch_p, fcW, fcbr):
    grid = _NPAD // _BM

    def body(*refs):
        hr = refs[0:4]
        bat, fwr, fbr = refs[4], refs[5], refs[6]
        out, sums, cnts = refs[7], refs[8], refs[9]
        i = pl.program_id(0)

        @pl.when(i == 0)
        def _():
            sums[...] = jnp.zeros_like(sums)
            cnts[...] = jnp.zeros_like(cnts)

        h = jnp.concatenate([hc[...] for hc in hr], axis=1)
        gids = lax.broadcasted_iota(jnp.int32, (1, _G), 1)
        mask = (bat[...] == gids).astype(jnp.float32)
        sums[...] += lax.dot_general(mask, h, (((0,), (0,)), ((), ())),
                                     preferred_element_type=jnp.float32)
        cnts[...] += lax.dot_general(
            mask, jnp.ones((_BM, 128), jnp.float32),
            (((0,), (0,)), ((), ())), preferred_element_type=jnp.float32)

        @pl.when(i == grid - 1)
        def _():
            pooled = sums[...] / jnp.maximum(cnts[...][:, :1], 1.0)
            logits = lax.dot_general(pooled, fwr[...],
                                     (((1,), (0,)), ((), ())),
                                     preferred_element_type=jnp.float32)
            logits = logits + fbr[...]
            m = jnp.max(logits, axis=1, keepdims=True)
            e = jnp.exp(logits - m)
            out[...] = (logits - m) - jnp.log(jnp.sum(e, axis=1,
                                                      keepdims=True))

    cspec = pl.BlockSpec((_BM, _FC), lambda i: (i, 0))
    return pl.pallas_call(
        body,
        grid=(grid,),
        in_specs=[cspec] * 4 + [
            pl.BlockSpec((_BM, 1), lambda i: (i, 0)),
            pl.BlockSpec((512, 64), lambda i: (0, 0)),
            pl.BlockSpec((1, 64), lambda i: (0, 0))],
        out_specs=pl.BlockSpec((_G, 64), lambda i: (0, 0)),
        out_shape=jax.ShapeDtypeStruct((_G, 64), jnp.float32),
        scratch_shapes=[pltpu.VMEM((_G, 512), jnp.float32),
                        pltpu.VMEM((_G, 128), jnp.float32)],
    )(*h2s, batch_p, fcW, fcbr)


def kernel(x, edge_index, batch, W1, b1, W2, b2, fc_W, fc_b):
    src = edge_index[0]
    dst = edge_index[1]
    pe = _EPAD - _E
    aux = jnp.arange(pe, dtype=jnp.int32)
    src_p = jnp.concatenate([src, aux % _N])
    dst_p = jnp.concatenate([dst, _N + (aux % 16)])
    src_r = src_p.reshape(_NS, _NSTR, 128)
    dst_r32 = dst_p.reshape(_NC * _NS, _DSTR, 128)
    mod16 = jnp.arange(_EPAD, dtype=jnp.int32) % 16
    dummy = _HALF + mod16
    dst_lo = jnp.where(dst_p < _HALF, dst_p, dummy)
    dst_hi = jnp.where((dst_p >= _HALF) & (dst_p < _N), dst_p - _HALF, dummy)
    dst_r = jnp.stack([dst_lo.reshape(_NS, _NSTR, 128),
                       dst_hi.reshape(_NS, _NSTR, 128)]
                      ).reshape(_NC * _NS, _NSTR, 128)
    zeros1 = jnp.zeros((_RPS,), jnp.float32)
    zeros2 = jnp.zeros((_HRPS, _FC), jnp.float32)
    x_pad = jnp.pad(x, ((0, _NPAD - _N), (0, 0)))
    batch_p = jnp.concatenate(
        [batch, jnp.full((_NPAD - _N,), _G, jnp.int32)]).reshape(_NPAD, 1)

    deg0, deg1 = _deg_kernel(dst_r32, zeros1)
    dinv, t10, t11 = _tc_scale(x_pad, deg0.reshape(_NPAD, 1),
                               deg1.reshape(_NPAD, 1))
    tz = jnp.zeros((_NPAD, _FC), jnp.float32)
    Wstack = jnp.stack([jnp.pad(W1, ((0, 256), (0, 0))), W2])
    bstack = jnp.stack([b1.reshape(1, 512), b2.reshape(1, 512)])

    def layer(ts, wb):
        W, br = wb
        aggs = _agg4(*ts, src_r, dst_r, zeros2)
        outs = _tc_layer(aggs, ts, dinv, W, br)
        return tuple(outs[4:8]), tuple(outs[0:4])

    _, hs = lax.scan(layer, (t10, t11, tz, tz), (Wstack, bstack))
    h2s = tuple(h[1] for h in hs)
    return _tc_head(h2s, batch_p, fc_W, fc_b.reshape(1, 64))


# speedup vs baseline: 10.9884x; 1.1994x over previous
import functools

import jax
import jax.numpy as jnp
from jax import lax
from jax.experimental import pallas as pl
from jax.experimental.pallas import tpu as pltpu
from jax.experimental.pallas import tpu_sc as plsc

_NC, _NS = 2, 16
_N = 10000
_E = 160000
_NPAD = 10240
_RPS = _NPAD // _NS
_EPAD = 163840
_NSTR = 80
_DSTR = 40
_FC = 128
_BM = 1024
_G = 16
_HALF = _NPAD // 2
_HRPS = _HALF // _NS
_ACCR = _HALF + 128


def _sc_mesh():
    return plsc.VectorSubcoreMesh(
        core_axis_name="c", subcore_axis_name="s",
        num_cores=_NC, num_subcores=_NS)


@functools.partial(
    pl.kernel,
    out_type=(jax.ShapeDtypeStruct((_NPAD,), jnp.float32),
              jax.ShapeDtypeStruct((_NPAD,), jnp.float32)),
    mesh=_sc_mesh(),
    scratch_types=[
        pltpu.VMEM((_DSTR, 128), jnp.int32),
        pltpu.VMEM((128,), jnp.float32),
        pltpu.VMEM_SHARED((_NPAD,), jnp.float32),
        pltpu.SemaphoreType.DMA,
    ],
)
def _deg_kernel(dst_hbm, zeros_hbm, deg0_hbm, deg1_hbm, dst_v, ones_v, acc, sem):
    c = lax.axis_index("c")
    s = lax.axis_index("s")
    w = c * _NS + s
    pltpu.sync_copy(dst_hbm.at[w], dst_v)
    for i in range(8):
        ones_v[pl.ds(16 * i, 16)] = jnp.ones((16,), jnp.float32)
    pltpu.sync_copy(zeros_hbm, acc.at[pl.ds(s * _RPS, _RPS)])
    plsc.subcore_barrier()

    def blk(o, carry):
        def fire(i, carry2):
            pltpu.async_copy(ones_v, acc.at[dst_v.at[o * 8 + i]], sem, add=True)
            return carry2

        lax.fori_loop(0, 8, fire, 0)

        def drain(i, carry2):
            pltpu.make_async_copy(ones_v, acc.at[dst_v.at[0]], sem).wait()
            return carry2

        lax.fori_loop(0, 8, drain, 0)
        return carry

    lax.fori_loop(0, _DSTR // 8, blk, 0)
    plsc.subcore_barrier()

    @pl.when(c == 0)
    def _():
        pltpu.sync_copy(acc.at[pl.ds(s * _RPS, _RPS)],
                        deg0_hbm.at[pl.ds(s * _RPS, _RPS)])

    @pl.when(c == 1)
    def _():
        pltpu.sync_copy(acc.at[pl.ds(s * _RPS, _RPS)],
                        deg1_hbm.at[pl.ds(s * _RPS, _RPS)])


def _run_chunk(t, o, src_v, dst_v, bufs, acc, zeros_hbm, gsems, ssems, s,
               base, do_edges):
    pltpu.sync_copy(zeros_hbm, acc.at[pl.ds(s * _HRPS, _HRPS)])
    plsc.subcore_barrier()

    @pl.when(do_edges)
    def _():
        for j in range(3):
            pltpu.async_copy(t.at[src_v.at[j]], bufs[j], gsems[j])

        def body(i, carry):
            for b in range(4):
                j = 4 * i + b
                pltpu.make_async_copy(t.at[src_v.at[0]], bufs[b],
                                      gsems[b]).wait()
                pltpu.async_copy(bufs[b], acc.at[dst_v.at[j]], ssems[b],
                                 add=True)
                pb = (b - 1) % 4
                if b == 0:
                    @pl.when(i >= 1)
                    def _():
                        pltpu.make_async_copy(
                            bufs[pb], acc.at[dst_v.at[0]], ssems[pb]).wait()
                else:
                    pltpu.make_async_copy(
                        bufs[pb], acc.at[dst_v.at[0]], ssems[pb]).wait()
                gb = (b + 3) % 4
                if b == 0:
                    pltpu.async_copy(t.at[src_v.at[j + 3]], bufs[gb],
                                     gsems[gb])
                else:
                    @pl.when(i < _NSTR // 4 - 1)
                    def _():
                        pltpu.async_copy(t.at[src_v.at[j + 3]], bufs[gb],
                                         gsems[gb])
            return carry

        lax.fori_loop(0, _NSTR // 4, body, 0)
        pltpu.make_async_copy(bufs[3], acc.at[dst_v.at[0]], ssems[3]).wait()

    plsc.subcore_barrier()
    pltpu.sync_copy(acc.at[pl.ds(s * _HRPS, _HRPS)],
                    o.at[pl.ds(base + s * _HRPS, _HRPS)])
    plsc.subcore_barrier()


_AGG_SCRATCH = ([pltpu.VMEM((_NSTR, 128), jnp.int32),
                 pltpu.VMEM((_NSTR, 128), jnp.int32),
                 pltpu.VMEM((16,), jnp.int32)]
                + [pltpu.VMEM((128, _FC), jnp.float32) for _ in range(4)]
                + [pltpu.VMEM_SHARED((_ACCR, _FC), jnp.float32)]
                + [pltpu.SemaphoreType.DMA for _ in range(8)])


@functools.partial(
    pl.kernel,
    out_type=tuple(jax.ShapeDtypeStruct((_NPAD, _FC), jnp.float32)
                   for _ in range(4)),
    mesh=_sc_mesh(),
    scratch_types=_AGG_SCRATCH,
)
def _agg4(t0, t1, t2, t3, src_hbm, dst_hbm, act_hbm, zeros_hbm,
          o0, o1, o2, o3,
          src_v, dst_v, act_v, b0, b1, b2, b3, acc,
          g0, g1, g2, g3, s0, s1, s2, s3):
    tables = (t0, t1, t2, t3)
    out_refs = (o0, o1, o2, o3)
    bufs = (b0, b1, b2, b3)
    gsems = (g0, g1, g2, g3)
    ssems = (s0, s1, s2, s3)
    c = lax.axis_index("c")
    s = lax.axis_index("s")
    w = c * _NS + s
    base = c * _HALF
    pltpu.sync_copy(src_hbm.at[s], src_v)
    pltpu.sync_copy(dst_hbm.at[w], dst_v)
    pltpu.sync_copy(act_hbm, act_v)
    a = act_v[pl.ds(0, 16)][0]
    for k in range(4):
        do_edges = (a > 0) if k >= 2 else (a > -1)
        _run_chunk(tables[k], out_refs[k], src_v, dst_v, bufs,
                   acc, zeros_hbm, gsems, ssems, s, base, do_edges)


def _tc_scale(x_pad, deg0, deg1):
    grid = _NPAD // _BM

    def body(x_ref, d0_ref, d1_ref, dinv_ref, t0_ref, t1_ref):
        deg = d0_ref[...] + d1_ref[...] + 1.0
        dinv = lax.rsqrt(deg)
        dinv_ref[...] = dinv
        t = x_ref[...] * dinv
        t0_ref[...] = t[:, :_FC]
        t1_ref[...] = t[:, _FC:]

    return pl.pallas_call(
        body,
        grid=(grid,),
        in_specs=[pl.BlockSpec((_BM, 256), lambda i: (i, 0)),
                  pl.BlockSpec((_BM, 1), lambda i: (i, 0)),
                  pl.BlockSpec((_BM, 1), lambda i: (i, 0))],
        out_specs=[pl.BlockSpec((_BM, 1), lambda i: (i, 0))]
        + [pl.BlockSpec((_BM, _FC), lambda i: (i, 0))] * 2,
        out_shape=[jax.ShapeDtypeStruct((_NPAD, 1), jnp.float32)]
        + [jax.ShapeDtypeStruct((_NPAD, _FC), jnp.float32)] * 2,
    )(x_pad, deg0, deg1)


def _tc_layer(aggs, ts, dinv, W, br):
    grid = _NPAD // _BM

    def body(*refs):
        ar = refs[0:4]
        tr = refs[4:8]
        dr, wr, brr = refs[8], refs[9], refs[10]
        houts = refs[11:15]
        touts = refs[15:19]
        dv = dr[...]
        z = jnp.concatenate([a[...] + t[...] for a, t in zip(ar, tr)],
                            axis=1) * dv
        h = lax.dot_general(z, wr[...], (((1,), (0,)), ((), ())),
                            preferred_element_type=jnp.float32)
        h = jnp.maximum(h + brr[...], 0.0)
        tn = h * dv
        for k in range(4):
            houts[k][...] = h[:, _FC * k:_FC * (k + 1)]
            touts[k][...] = tn[:, _FC * k:_FC * (k + 1)]

    cspec = pl.BlockSpec((_BM, _FC), lambda i: (i, 0))
    return pl.pallas_call(
        body,
        grid=(grid,),
        in_specs=[cspec] * 8 + [
            pl.BlockSpec((_BM, 1), lambda i: (i, 0)),
            pl.BlockSpec((512, 512), lambda i: (0, 0)),
            pl.BlockSpec((1, 512), lambda i: (0, 0))],
        out_specs=[cspec] * 8,
        out_shape=[jax.ShapeDtypeStruct((_NPAD, _FC), jnp.float32)
                   for _ in range(8)],
    )(*aggs, *ts, dinv, W, br)


def _tc_head(h2s, batch_p, fcW, fcbr):
    grid = _NPAD // _BM

    def body(*refs):
        hr = refs[0:4]
        bat, fwr, fbr = refs[4], refs[5], refs[6]
        out, sums, cnts = refs[7], refs[8], refs[9]
        i = pl.program_id(0)

        @pl.when(i == 0)
        def _():
            sums[...] = jnp.zeros_like(sums)
            cnts[...] = jnp.zeros_like(cnts)

        h = jnp.concatenate([hc[...] for hc in hr], axis=1)
        gids = lax.broadcasted_iota(jnp.int32, (1, _G), 1)
        mask = (bat[...] == gids).astype(jnp.float32)
        sums[...] += lax.dot_general(mask, h, (((0,), (0,)), ((), ())),
                                     preferred_element_type=jnp.float32)
        cnts[...] += lax.dot_general(
            mask, jnp.ones((_BM, 128), jnp.float32),
            (((0,), (0,)), ((), ())), preferred_element_type=jnp.float32)

        @pl.when(i == grid - 1)
        def _():
            pooled = sums[...] / jnp.maximum(cnts[...][:, :1], 1.0)
            logits = lax.dot_general(pooled, fwr[...],
                                     (((1,), (0,)), ((), ())),
                                     preferred_element_type=jnp.float32)
            logits = logits + fbr[...]
            m = jnp.max(logits, axis=1, keepdims=True)
            e = jnp.exp(logits - m)
            out[...] = (logits - m) - jnp.log(jnp.sum(e, axis=1,
                                                      keepdims=True))

    cspec = pl.BlockSpec((_BM, _FC), lambda i: (i, 0))
    return pl.pallas_call(
        body,
        grid=(grid,),
        in_specs=[cspec] * 4 + [
            pl.BlockSpec((_BM, 1), lambda i: (i, 0)),
            pl.BlockSpec((512, 64), lambda i: (0, 0)),
            pl.BlockSpec((1, 64), lambda i: (0, 0))],
        out_specs=pl.BlockSpec((_G, 64), lambda i: (0, 0)),
        out_shape=jax.ShapeDtypeStruct((_G, 64), jnp.float32),
        scratch_shapes=[pltpu.VMEM((_G, 512), jnp.float32),
                        pltpu.VMEM((_G, 128), jnp.float32)],
    )(*h2s, batch_p, fcW, fcbr)


def kernel(x, edge_index, batch, W1, b1, W2, b2, fc_W, fc_b):
    src = edge_index[0]
    dst = edge_index[1]
    pe = _EPAD - _E
    aux = jnp.arange(pe, dtype=jnp.int32)
    src_p = jnp.concatenate([src, aux % _N])
    dst_p = jnp.concatenate([dst, _N + (aux % 16)])
    src_r = src_p.reshape(_NS, _NSTR, 128)
    dst_r32 = dst_p.reshape(_NC * _NS, _DSTR, 128)
    dummy = _HALF + (jnp.arange(_EPAD, dtype=jnp.int32) % 128)
    dst_lo = jnp.where(dst_p < _HALF, dst_p, dummy)
    dst_hi = jnp.where((dst_p >= _HALF) & (dst_p < _N), dst_p - _HALF, dummy)
    dst_r = jnp.stack([dst_lo.reshape(_NS, _NSTR, 128),
                       dst_hi.reshape(_NS, _NSTR, 128)]
                      ).reshape(_NC * _NS, _NSTR, 128)
    zeros1 = jnp.zeros((_RPS,), jnp.float32)
    zeros2 = jnp.zeros((_HRPS, _FC), jnp.float32)
    x_pad = jnp.pad(x, ((0, _NPAD - _N), (0, 0)))
    batch_p = jnp.concatenate(
        [batch, jnp.full((_NPAD - _N,), _G, jnp.int32)]).reshape(_NPAD, 1)

    deg0, deg1 = _deg_kernel(dst_r32, zeros1)
    dinv, t10, t11 = _tc_scale(x_pad, deg0.reshape(_NPAD, 1),
                               deg1.reshape(_NPAD, 1))
    tz = jnp.zeros((_NPAD, _FC), jnp.float32)
    Wstack = jnp.stack([jnp.pad(W1, ((0, 256), (0, 0))), W2])
    bstack = jnp.stack([b1.reshape(1, 512), b2.reshape(1, 512)])

    acts = jnp.array([[0] * 16, [1] * 16], dtype=jnp.int32)

    def layer(ts, wb):
        W, br, act = wb
        aggs = _agg4(*ts, src_r, dst_r, act, zeros2)
        outs = _tc_layer(aggs, ts, dinv, W, br)
        return tuple(outs[4:8]), tuple(outs[0:4])

    _, hs = lax.scan(layer, (t10, t11, tz, tz), (Wstack, bstack, acts))
    h2s = tuple(h[1] for h in hs)
    return _tc_head(h2s, batch_p, fc_W, fc_b.reshape(1, 64))


# speedup vs baseline: 11.9339x; 1.0860x over previous
import functools

import jax
import jax.numpy as jnp
from jax import lax
from jax.experimental import pallas as pl
from jax.experimental.pallas import tpu as pltpu
from jax.experimental.pallas import tpu_sc as plsc

_NC, _NS = 2, 16
_N = 10000
_E = 160000
_NPAD = 10240
_RPS = _NPAD // _NS
_EPAD = 163840
_NSTR = 80
_DSTR = 40
_FC = 128
_BM = 1024
_G = 16
_HALF = _NPAD // 2
_HRPS = _HALF // _NS
_ACCR = _HALF + 128


def _sc_mesh():
    return plsc.VectorSubcoreMesh(
        core_axis_name="c", subcore_axis_name="s",
        num_cores=_NC, num_subcores=_NS)


@functools.partial(
    pl.kernel,
    out_type=(jax.ShapeDtypeStruct((_NPAD,), jnp.float32),
              jax.ShapeDtypeStruct((_NPAD,), jnp.float32)),
    mesh=_sc_mesh(),
    scratch_types=[
        pltpu.VMEM((_DSTR, 128), jnp.int32),
        pltpu.VMEM((128,), jnp.float32),
        pltpu.VMEM_SHARED((_NPAD,), jnp.float32),
        pltpu.SemaphoreType.DMA,
    ],
)
def _deg_kernel(dst_hbm, zeros_hbm, deg0_hbm, deg1_hbm, dst_v, ones_v, acc, sem):
    c = lax.axis_index("c")
    s = lax.axis_index("s")
    w = c * _NS + s
    pltpu.sync_copy(dst_hbm.at[w], dst_v)
    for i in range(8):
        ones_v[pl.ds(16 * i, 16)] = jnp.ones((16,), jnp.float32)
    pltpu.sync_copy(zeros_hbm, acc.at[pl.ds(s * _RPS, _RPS)])
    plsc.subcore_barrier()

    def blk(o, carry):
        def fire(i, carry2):
            pltpu.async_copy(ones_v, acc.at[dst_v.at[o * 8 + i]], sem, add=True)
            return carry2

        lax.fori_loop(0, 8, fire, 0)

        def drain(i, carry2):
            pltpu.make_async_copy(ones_v, acc.at[dst_v.at[0]], sem).wait()
            return carry2

        lax.fori_loop(0, 8, drain, 0)
        return carry

    lax.fori_loop(0, _DSTR // 8, blk, 0)
    plsc.subcore_barrier()

    @pl.when(c == 0)
    def _():
        pltpu.sync_copy(acc.at[pl.ds(s * _RPS, _RPS)],
                        deg0_hbm.at[pl.ds(s * _RPS, _RPS)])

    @pl.when(c == 1)
    def _():
        pltpu.sync_copy(acc.at[pl.ds(s * _RPS, _RPS)],
                        deg1_hbm.at[pl.ds(s * _RPS, _RPS)])


def _run_chunk(t, o, src_v, dst_v, bufs, acc, zeros_hbm, gsems, ssems, s,
               base, do_edges):
    pltpu.sync_copy(zeros_hbm, acc.at[pl.ds(s * _HRPS, _HRPS)])
    plsc.subcore_barrier()

    @pl.when(do_edges)
    def _():
        for j in range(3):
            pltpu.async_copy(t.at[src_v.at[j]], bufs[j], gsems[j])

        def body(i, carry):
            for b in range(4):
                j = 4 * i + b
                pltpu.make_async_copy(t.at[src_v.at[0]], bufs[b],
                                      gsems[b]).wait()
                pltpu.async_copy(bufs[b], acc.at[dst_v.at[j]], ssems[b],
                                 add=True)
                pb = (b - 1) % 4
                if b == 0:
                    @pl.when(i >= 1)
                    def _():
                        pltpu.make_async_copy(
                            bufs[pb], acc.at[dst_v.at[0]], ssems[pb]).wait()
                else:
                    pltpu.make_async_copy(
                        bufs[pb], acc.at[dst_v.at[0]], ssems[pb]).wait()
                gb = (b + 3) % 4
                if b == 0:
                    pltpu.async_copy(t.at[src_v.at[j + 3]], bufs[gb],
                                     gsems[gb])
                else:
                    @pl.when(i < _NSTR // 4 - 1)
                    def _():
                        pltpu.async_copy(t.at[src_v.at[j + 3]], bufs[gb],
                                         gsems[gb])
            return carry

        lax.fori_loop(0, _NSTR // 4, body, 0)
        pltpu.make_async_copy(bufs[3], acc.at[dst_v.at[0]], ssems[3]).wait()

    plsc.subcore_barrier()
    pltpu.sync_copy(acc.at[pl.ds(s * _HRPS, _HRPS)],
                    o.at[pl.ds(base + s * _HRPS, _HRPS)])
    plsc.subcore_barrier()


_AGG_SCRATCH = ([pltpu.VMEM((_NSTR, 128), jnp.int32),
                 pltpu.VMEM((_NSTR, 128), jnp.int32),
                 pltpu.VMEM((16,), jnp.int32)]
                + [pltpu.VMEM((128, _FC), jnp.float32) for _ in range(4)]
                + [pltpu.VMEM_SHARED((_ACCR, _FC), jnp.float32)]
                + [pltpu.SemaphoreType.DMA for _ in range(8)])


@functools.partial(
    pl.kernel,
    out_type=tuple(jax.ShapeDtypeStruct((_NPAD, _FC), jnp.float32)
                   for _ in range(4)),
    mesh=_sc_mesh(),
    scratch_types=_AGG_SCRATCH,
)
def _agg4(t0, t1, t2, t3, src_hbm, dst_hbm, act_hbm, zeros_hbm,
          o0, o1, o2, o3,
          src_v, dst_v, act_v, b0, b1, b2, b3, acc,
          g0, g1, g2, g3, s0, s1, s2, s3):
    tables = (t0, t1, t2, t3)
    out_refs = (o0, o1, o2, o3)
    bufs = (b0, b1, b2, b3)
    gsems = (g0, g1, g2, g3)
    ssems = (s0, s1, s2, s3)
    c = lax.axis_index("c")
    s = lax.axis_index("s")
    w = c * _NS + s
    base = c * _HALF
    pltpu.sync_copy(src_hbm.at[s], src_v)
    pltpu.sync_copy(dst_hbm.at[w], dst_v)
    pltpu.sync_copy(act_hbm, act_v)
    a = act_v[pl.ds(0, 16)][0]
    for k in range(4):
        do_edges = (a > 0) if k >= 2 else (a > -1)
        _run_chunk(tables[k], out_refs[k], src_v, dst_v, bufs,
                   acc, zeros_hbm, gsems, ssems, s, base, do_edges)


def _tc_scale(x_pad, deg0, deg1):
    grid = _NPAD // _BM

    def body(x_ref, d0_ref, d1_ref, dinv_ref, t0_ref, t1_ref):
        deg = d0_ref[...] + d1_ref[...] + 1.0
        dinv = lax.rsqrt(deg)
        dinv_ref[...] = dinv
        t = x_ref[...] * dinv
        t0_ref[...] = t[:, :_FC]
        t1_ref[...] = t[:, _FC:]

    return pl.pallas_call(
        body,
        grid=(grid,),
        in_specs=[pl.BlockSpec((_BM, 256), lambda i: (i, 0)),
                  pl.BlockSpec((_BM, 1), lambda i: (i, 0)),
                  pl.BlockSpec((_BM, 1), lambda i: (i, 0))],
        out_specs=[pl.BlockSpec((_BM, 1), lambda i: (i, 0))]
        + [pl.BlockSpec((_BM, _FC), lambda i: (i, 0))] * 2,
        out_shape=[jax.ShapeDtypeStruct((_NPAD, 1), jnp.float32)]
        + [jax.ShapeDtypeStruct((_NPAD, _FC), jnp.float32)] * 2,
    )(x_pad, deg0, deg1)


def _tc_layer(aggs, ts, dinv, W, br):
    grid = _NPAD // _BM

    def body(*refs):
        ar = refs[0:4]
        tr = refs[4:8]
        dr, wr, brr = refs[8], refs[9], refs[10]
        touts = refs[11:15]
        dv = dr[...]
        z = jnp.concatenate([a[...] + t[...] for a, t in zip(ar, tr)],
                            axis=1) * dv
        h = lax.dot_general(z, wr[...], (((1,), (0,)), ((), ())),
                            preferred_element_type=jnp.float32)
        tn = jnp.maximum(h + brr[...], 0.0) * dv
        for k in range(4):
            touts[k][...] = tn[:, _FC * k:_FC * (k + 1)]

    cspec = pl.BlockSpec((_BM, _FC), lambda i: (i, 0))
    return pl.pallas_call(
        body,
        grid=(grid,),
        in_specs=[cspec] * 8 + [
            pl.BlockSpec((_BM, 1), lambda i: (i, 0)),
            pl.BlockSpec((512, 512), lambda i: (0, 0)),
            pl.BlockSpec((1, 512), lambda i: (0, 0))],
        out_specs=[cspec] * 4,
        out_shape=[jax.ShapeDtypeStruct((_NPAD, _FC), jnp.float32)
                   for _ in range(4)],
    )(*aggs, *ts, dinv, W, br)


def _tc_head(h2s, dinv, batch_p, fcW, fcbr):
    grid = _NPAD // _BM

    def body(*refs):
        hr = refs[0:4]
        dr, bat, fwr, fbr = refs[4], refs[5], refs[6], refs[7]
        out, sums, cnts = refs[8], refs[9], refs[10]
        i = pl.program_id(0)

        @pl.when(i == 0)
        def _():
            sums[...] = jnp.zeros_like(sums)
            cnts[...] = jnp.zeros_like(cnts)

        h = jnp.concatenate([hc[...] for hc in hr], axis=1) / dr[...]
        gids = lax.broadcasted_iota(jnp.int32, (1, _G), 1)
        mask = (bat[...] == gids).astype(jnp.float32)
        sums[...] += lax.dot_general(mask, h, (((0,), (0,)), ((), ())),
                                     preferred_element_type=jnp.float32)
        cnts[...] += lax.dot_general(
            mask, jnp.ones((_BM, 128), jnp.float32),
            (((0,), (0,)), ((), ())), preferred_element_type=jnp.float32)

        @pl.when(i == grid - 1)
        def _():
            pooled = sums[...] / jnp.maximum(cnts[...][:, :1], 1.0)
            logits = lax.dot_general(pooled, fwr[...],
                                     (((1,), (0,)), ((), ())),
                                     preferred_element_type=jnp.float32)
            logits = logits + fbr[...]
            m = jnp.max(logits, axis=1, keepdims=True)
            e = jnp.exp(logits - m)
            out[...] = (logits - m) - jnp.log(jnp.sum(e, axis=1,
                                                      keepdims=True))

    cspec = pl.BlockSpec((_BM, _FC), lambda i: (i, 0))
    return pl.pallas_call(
        body,
        grid=(grid,),
        in_specs=[cspec] * 4 + [
            pl.BlockSpec((_BM, 1), lambda i: (i, 0)),
            pl.BlockSpec((_BM, 1), lambda i: (i, 0)),
            pl.BlockSpec((512, 64), lambda i: (0, 0)),
            pl.BlockSpec((1, 64), lambda i: (0, 0))],
        out_specs=pl.BlockSpec((_G, 64), lambda i: (0, 0)),
        out_shape=jax.ShapeDtypeStruct((_G, 64), jnp.float32),
        scratch_shapes=[pltpu.VMEM((_G, 512), jnp.float32),
                        pltpu.VMEM((_G, 128), jnp.float32)],
    )(*h2s, dinv, batch_p, fcW, fcbr)


def kernel(x, edge_index, batch, W1, b1, W2, b2, fc_W, fc_b):
    src = edge_index[0]
    dst = edge_index[1]
    pe = _EPAD - _E
    aux = jnp.arange(pe, dtype=jnp.int32)
    src_p = jnp.concatenate([src, aux % _N])
    dst_p = jnp.concatenate([dst, _N + (aux % 16)])
    src_r = src_p.reshape(_NS, _NSTR, 128)
    dst_r32 = dst_p.reshape(_NC * _NS, _DSTR, 128)
    dummy = _HALF + (jnp.arange(_EPAD, dtype=jnp.int32) % 128)
    dst_lo = jnp.where(dst_p < _HALF, dst_p, dummy)
    dst_hi = jnp.where((dst_p >= _HALF) & (dst_p < _N), dst_p - _HALF, dummy)
    dst_r = jnp.stack([dst_lo.reshape(_NS, _NSTR, 128),
                       dst_hi.reshape(_NS, _NSTR, 128)]
                      ).reshape(_NC * _NS, _NSTR, 128)
    zeros1 = jnp.zeros((_RPS,), jnp.float32)
    zeros2 = jnp.zeros((_HRPS, _FC), jnp.float32)
    x_pad = jnp.pad(x, ((0, _NPAD - _N), (0, 0)))
    batch_p = jnp.concatenate(
        [batch, jnp.full((_NPAD - _N,), _G, jnp.int32)]).reshape(_NPAD, 1)

    deg0, deg1 = _deg_kernel(dst_r32, zeros1)
    dinv, t10, t11 = _tc_scale(x_pad, deg0.reshape(_NPAD, 1),
                               deg1.reshape(_NPAD, 1))
    tz = jnp.zeros((_NPAD, _FC), jnp.float32)
    Wstack = jnp.stack([jnp.pad(W1, ((0, 256), (0, 0))), W2])
    bstack = jnp.stack([b1.reshape(1, 512), b2.reshape(1, 512)])

    acts = jnp.array([[0] * 16, [1] * 16], dtype=jnp.int32)

    def layer(ts, wb):
        W, br, act = wb
        aggs = _agg4(*ts, src_r, dst_r, act, zeros2)
        outs = _tc_layer(aggs, ts, dinv, W, br)
        return tuple(outs), None

    t3, _ = lax.scan(layer, (t10, t11, tz, tz), (Wstack, bstack, acts))
    return _tc_head(t3, dinv, batch_p, fc_W, fc_b.reshape(1, 64))


# speedup vs baseline: 12.0071x; 1.0061x over previous
import functools

import jax
import jax.numpy as jnp
from jax import lax
from jax.experimental import pallas as pl
from jax.experimental.pallas import tpu as pltpu
from jax.experimental.pallas import tpu_sc as plsc

_NC, _NS = 2, 16
_N = 10000
_E = 160000
_NPAD = 10240
_RPS = _NPAD // _NS
_EPAD = 163840
_NSTR = 80
_DSTR = 40
_FC = 128
_BM = 2048
_G = 16
_HALF = _NPAD // 2
_HRPS = _HALF // _NS
_ACCR = _HALF + 128


def _sc_mesh():
    return plsc.VectorSubcoreMesh(
        core_axis_name="c", subcore_axis_name="s",
        num_cores=_NC, num_subcores=_NS)


@functools.partial(
    pl.kernel,
    out_type=(jax.ShapeDtypeStruct((_NPAD,), jnp.float32),
              jax.ShapeDtypeStruct((_NPAD,), jnp.float32)),
    mesh=_sc_mesh(),
    scratch_types=[
        pltpu.VMEM((_DSTR, 128), jnp.int32),
        pltpu.VMEM((128,), jnp.float32),
        pltpu.VMEM_SHARED((_NPAD,), jnp.float32),
        pltpu.SemaphoreType.DMA,
    ],
)
def _deg_kernel(dst_hbm, zeros_hbm, deg0_hbm, deg1_hbm, dst_v, ones_v, acc, sem):
    c = lax.axis_index("c")
    s = lax.axis_index("s")
    w = c * _NS + s
    pltpu.sync_copy(dst_hbm.at[w], dst_v)
    for i in range(8):
        ones_v[pl.ds(16 * i, 16)] = jnp.ones((16,), jnp.float32)
    pltpu.sync_copy(zeros_hbm, acc.at[pl.ds(s * _RPS, _RPS)])
    plsc.subcore_barrier()

    def blk(o, carry):
        def fire(i, carry2):
            pltpu.async_copy(ones_v, acc.at[dst_v.at[o * 8 + i]], sem, add=True)
            return carry2

        lax.fori_loop(0, 8, fire, 0)

        def drain(i, carry2):
            pltpu.make_async_copy(ones_v, acc.at[dst_v.at[0]], sem).wait()
            return carry2

        lax.fori_loop(0, 8, drain, 0)
        return carry

    lax.fori_loop(0, _DSTR // 8, blk, 0)
    plsc.subcore_barrier()

    @pl.when(c == 0)
    def _():
        pltpu.sync_copy(acc.at[pl.ds(s * _RPS, _RPS)],
                        deg0_hbm.at[pl.ds(s * _RPS, _RPS)])

    @pl.when(c == 1)
    def _():
        pltpu.sync_copy(acc.at[pl.ds(s * _RPS, _RPS)],
                        deg1_hbm.at[pl.ds(s * _RPS, _RPS)])


def _run_chunk(t, o, src_v, dst_v, bufs, acc, zeros_hbm, gsems, ssems, s,
               base, do_edges):
    pltpu.sync_copy(zeros_hbm, acc.at[pl.ds(s * _HRPS, _HRPS)])
    plsc.subcore_barrier()

    @pl.when(do_edges)
    def _():
        for j in range(3):
            pltpu.async_copy(t.at[src_v.at[j]], bufs[j], gsems[j])

        def body(i, carry):
            for b in range(4):
                j = 4 * i + b
                pltpu.make_async_copy(t.at[src_v.at[0]], bufs[b],
                                      gsems[b]).wait()
                pltpu.async_copy(bufs[b], acc.at[dst_v.at[j]], ssems[b],
                                 add=True)
                pb = (b - 1) % 4
                if b == 0:
                    @pl.when(i >= 1)
                    def _():
                        pltpu.make_async_copy(
                            bufs[pb], acc.at[dst_v.at[0]], ssems[pb]).wait()
                else:
                    pltpu.make_async_copy(
                        bufs[pb], acc.at[dst_v.at[0]], ssems[pb]).wait()
                gb = (b + 3) % 4
                if b == 0:
                    pltpu.async_copy(t.at[src_v.at[j + 3]], bufs[gb],
                                     gsems[gb])
                else:
                    @pl.when(i < _NSTR // 4 - 1)
                    def _():
                        pltpu.async_copy(t.at[src_v.at[j + 3]], bufs[gb],
                                         gsems[gb])
            return carry

        lax.fori_loop(0, _NSTR // 4, body, 0)
        pltpu.make_async_copy(bufs[3], acc.at[dst_v.at[0]], ssems[3]).wait()

    plsc.subcore_barrier()
    pltpu.sync_copy(acc.at[pl.ds(s * _HRPS, _HRPS)],
                    o.at[pl.ds(base + s * _HRPS, _HRPS)])
    plsc.subcore_barrier()


_AGG_SCRATCH = ([pltpu.VMEM((_NSTR, 128), jnp.int32),
                 pltpu.VMEM((_NSTR, 128), jnp.int32),
                 pltpu.VMEM((16,), jnp.int32)]
                + [pltpu.VMEM((128, _FC), jnp.float32) for _ in range(4)]
                + [pltpu.VMEM_SHARED((_ACCR, _FC), jnp.float32)]
                + [pltpu.SemaphoreType.DMA for _ in range(8)])


@functools.partial(
    pl.kernel,
    out_type=tuple(jax.ShapeDtypeStruct((_NPAD, _FC), jnp.float32)
                   for _ in range(4)),
    mesh=_sc_mesh(),
    scratch_types=_AGG_SCRATCH,
)
def _agg4(t0, t1, t2, t3, src_hbm, dst_hbm, act_hbm, zeros_hbm,
          o0, o1, o2, o3,
          src_v, dst_v, act_v, b0, b1, b2, b3, acc,
          g0, g1, g2, g3, s0, s1, s2, s3):
    tables = (t0, t1, t2, t3)
    out_refs = (o0, o1, o2, o3)
    bufs = (b0, b1, b2, b3)
    gsems = (g0, g1, g2, g3)
    ssems = (s0, s1, s2, s3)
    c = lax.axis_index("c")
    s = lax.axis_index("s")
    w = c * _NS + s
    base = c * _HALF
    pltpu.sync_copy(src_hbm.at[s], src_v)
    pltpu.sync_copy(dst_hbm.at[w], dst_v)
    pltpu.sync_copy(act_hbm, act_v)
    a = act_v[pl.ds(0, 16)][0]
    for k in range(4):
        do_edges = (a > 0) if k >= 2 else (a > -1)
        _run_chunk(tables[k], out_refs[k], src_v, dst_v, bufs,
                   acc, zeros_hbm, gsems, ssems, s, base, do_edges)


def _tc_scale(x_pad, deg0, deg1):
    grid = _NPAD // _BM

    def body(x_ref, d0_ref, d1_ref, dinv_ref, t0_ref, t1_ref):
        deg = d0_ref[...] + d1_ref[...] + 1.0
        dinv = lax.rsqrt(deg)
        dinv_ref[...] = dinv
        t = x_ref[...] * dinv
        t0_ref[...] = t[:, :_FC]
        t1_ref[...] = t[:, _FC:]

    return pl.pallas_call(
        body,
        grid=(grid,),
        in_specs=[pl.BlockSpec((_BM, 256), lambda i: (i, 0)),
                  pl.BlockSpec((_BM, 1), lambda i: (i, 0)),
                  pl.BlockSpec((_BM, 1), lambda i: (i, 0))],
        out_specs=[pl.BlockSpec((_BM, 1), lambda i: (i, 0))]
        + [pl.BlockSpec((_BM, _FC), lambda i: (i, 0))] * 2,
        out_shape=[jax.ShapeDtypeStruct((_NPAD, 1), jnp.float32)]
        + [jax.ShapeDtypeStruct((_NPAD, _FC), jnp.float32)] * 2,
    )(x_pad, deg0, deg1)


def _tc_layer(aggs, ts, dinv, W, br):
    grid = _NPAD // _BM

    def body(*refs):
        ar = refs[0:4]
        tr = refs[4:8]
        dr, wr, brr = refs[8], refs[9], refs[10]
        touts = refs[11:15]
        dv = dr[...]
        z = jnp.concatenate([a[...] + t[...] for a, t in zip(ar, tr)],
                            axis=1) * dv
        h = lax.dot_general(z, wr[...], (((1,), (0,)), ((), ())),
                            preferred_element_type=jnp.float32)
        tn = jnp.maximum(h + brr[...], 0.0) * dv
        for k in range(4):
            touts[k][...] = tn[:, _FC * k:_FC * (k + 1)]

    cspec = pl.BlockSpec((_BM, _FC), lambda i: (i, 0))
    return pl.pallas_call(
        body,
        grid=(grid,),
        in_specs=[cspec] * 8 + [
            pl.BlockSpec((_BM, 1), lambda i: (i, 0)),
            pl.BlockSpec((512, 512), lambda i: (0, 0)),
            pl.BlockSpec((1, 512), lambda i: (0, 0))],
        out_specs=[cspec] * 4,
        out_shape=[jax.ShapeDtypeStruct((_NPAD, _FC), jnp.float32)
                   for _ in range(4)],
    )(*aggs, *ts, dinv, W, br)


def _tc_head(h2s, dinv, batch_p, fcW, fcbr):
    grid = _NPAD // _BM

    def body(*refs):
        hr = refs[0:4]
        dr, bat, fwr, fbr = refs[4], refs[5], refs[6], refs[7]
        out, sums, cnts = refs[8], refs[9], refs[10]
        i = pl.program_id(0)

        @pl.when(i == 0)
        def _():
            sums[...] = jnp.zeros_like(sums)
            cnts[...] = jnp.zeros_like(cnts)

        h = jnp.concatenate([hc[...] for hc in hr], axis=1) / dr[...]
        gids = lax.broadcasted_iota(jnp.int32, (1, _G), 1)
        mask = (bat[...] == gids).astype(jnp.float32)
        sums[...] += lax.dot_general(mask, h, (((0,), (0,)), ((), ())),
                                     preferred_element_type=jnp.float32)
        cnts[...] += lax.dot_general(
            mask, jnp.ones((_BM, 128), jnp.float32),
            (((0,), (0,)), ((), ())), preferred_element_type=jnp.float32)

        @pl.when(i == grid - 1)
        def _():
            pooled = sums[...] / jnp.maximum(cnts[...][:, :1], 1.0)
            logits = lax.dot_general(pooled, fwr[...],
                                     (((1,), (0,)), ((), ())),
                                     preferred_element_type=jnp.float32)
            logits = logits + fbr[...]
            m = jnp.max(logits, axis=1, keepdims=True)
            e = jnp.exp(logits - m)
            out[...] = (logits - m) - jnp.log(jnp.sum(e, axis=1,
                                                      keepdims=True))

    cspec = pl.BlockSpec((_BM, _FC), lambda i: (i, 0))
    return pl.pallas_call(
        body,
        grid=(grid,),
        in_specs=[cspec] * 4 + [
            pl.BlockSpec((_BM, 1), lambda i: (i, 0)),
            pl.BlockSpec((_BM, 1), lambda i: (i, 0)),
            pl.BlockSpec((512, 64), lambda i: (0, 0)),
            pl.BlockSpec((1, 64), lambda i: (0, 0))],
        out_specs=pl.BlockSpec((_G, 64), lambda i: (0, 0)),
        out_shape=jax.ShapeDtypeStruct((_G, 64), jnp.float32),
        scratch_shapes=[pltpu.VMEM((_G, 512), jnp.float32),
                        pltpu.VMEM((_G, 128), jnp.float32)],
    )(*h2s, dinv, batch_p, fcW, fcbr)


def kernel(x, edge_index, batch, W1, b1, W2, b2, fc_W, fc_b):
    src = edge_index[0]
    dst = edge_index[1]
    pe = _EPAD - _E
    aux = jnp.arange(pe, dtype=jnp.int32)
    src_p = jnp.concatenate([src, aux % _N])
    dst_p = jnp.concatenate([dst, _N + (aux % 16)])
    src_r = src_p.reshape(_NS, _NSTR, 128)
    dst_r32 = dst_p.reshape(_NC * _NS, _DSTR, 128)
    dummy = _HALF + (jnp.arange(_EPAD, dtype=jnp.int32) % 128)
    dst_lo = jnp.where(dst_p < _HALF, dst_p, dummy)
    dst_hi = jnp.where((dst_p >= _HALF) & (dst_p < _N), dst_p - _HALF, dummy)
    dst_r = jnp.stack([dst_lo.reshape(_NS, _NSTR, 128),
                       dst_hi.reshape(_NS, _NSTR, 128)]
                      ).reshape(_NC * _NS, _NSTR, 128)
    zeros1 = jnp.zeros((_RPS,), jnp.float32)
    zeros2 = jnp.zeros((_HRPS, _FC), jnp.float32)
    x_pad = jnp.pad(x, ((0, _NPAD - _N), (0, 0)))
    batch_p = jnp.concatenate(
        [batch, jnp.full((_NPAD - _N,), _G, jnp.int32)]).reshape(_NPAD, 1)

    deg0, deg1 = _deg_kernel(dst_r32, zeros1)
    dinv, t10, t11 = _tc_scale(x_pad, deg0.reshape(_NPAD, 1),
                               deg1.reshape(_NPAD, 1))
    tz = jnp.zeros((_NPAD, _FC), jnp.float32)
    Wstack = jnp.stack([jnp.pad(W1, ((0, 256), (0, 0))), W2])
    bstack = jnp.stack([b1.reshape(1, 512), b2.reshape(1, 512)])

    acts = jnp.array([[0] * 16, [1] * 16], dtype=jnp.int32)

    def layer(ts, wb):
        W, br, act = wb
        aggs = _agg4(*ts, src_r, dst_r, act, zeros2)
        outs = _tc_layer(aggs, ts, dinv, W, br)
        return tuple(outs), None

    t3, _ = lax.scan(layer, (t10, t11, tz, tz), (Wstack, bstack, acts))
    return _tc_head(t3, dinv, batch_p, fc_W, fc_b.reshape(1, 64))
